# Initial kernel scaffold; baseline (speedup 1.0000x reference)
import jax
import jax.numpy as jnp
from jax.experimental import pallas as pl


def kernel(feature_map, dist, sin_angles, cos_angles, nd_sampling):
    raise NotImplementedError("write your pallas kernel here")


# speedup vs baseline: 2.2419x; 2.2419x over previous
import functools

import jax
import jax.numpy as jnp
from jax import lax
from jax.experimental import pallas as pl
from jax.experimental.pallas import tpu as pltpu
from jax.experimental.pallas import tpu_sc as plsc

B, C, H, W = 2, 384, 128, 128
K = 32
ND = C // K
HW = H * W
L = 16
MAGIC = float(1.5 * 2.0**23)


def _sc_body(fm, dist, consts, sf, sc,
             fm_v, out_v, idx_v, sc_v, dist_v, const_v):
    wid = lax.axis_index("s") * 2 + lax.axis_index("c")

    pltpu.sync_copy(consts, const_v)
    fm_v[pl.ds(HW, L)] = jnp.zeros((L,), jnp.float32)

    cbase = wid * (3 * L)
    sinv = const_v[pl.ds(cbase, L)]
    cosv = const_v[pl.ds(cbase + L, L)]
    scalev = const_v[pl.ds(cbase + 2 * L, L)]
    lanes = lax.iota(jnp.int32, L)
    lanes_f = lanes.astype(jnp.float32)

    for b in range(B):
        pltpu.sync_copy(dist.at[b], dist_v)

        def idx_body(y, _):
            yf = y.astype(jnp.float32)
            for j in range(W // L):
                off = y * W + j * L
                d = dist_v[pl.ds(off, L)]
                xf = lanes_f + jnp.float32(j * L)
                fx = xf + cosv * d
                fy = yf + sinv * d
                scx = (fx / jnp.float32(W - 1)) * 2.0 - 1.0
                scy = (fy / jnp.float32(H - 1)) * 2.0 - 1.0
                ixf = (scx + 1.0) * jnp.float32(W) * 0.5 - 0.5
                iyf = (scy + 1.0) * jnp.float32(H) * 0.5 - 0.5
                rx = (ixf + MAGIC) - MAGIC
                ry = (iyf + MAGIC) - MAGIC
                valid = ((rx >= 0.0) & (rx <= jnp.float32(W - 1))
                         & (ry >= 0.0) & (ry <= jnp.float32(H - 1)))
                cx = jnp.minimum(jnp.maximum(rx, 0.0), jnp.float32(W - 1))
                cy = jnp.minimum(jnp.maximum(ry, 0.0), jnp.float32(H - 1))
                idxf = cy * jnp.float32(W) + cx
                idxf = jnp.where(valid, idxf, jnp.float32(HW))
                idx_v[pl.ds(off, L)] = idxf.astype(jnp.int32)
                p = (off + lanes) * 2
                plsc.store_scatter(sc_v, [p], scx)
                plsc.store_scatter(sc_v, [p + 1], scy)
            return 0

        lax.fori_loop(0, H, idx_body, 0)
        pltpu.sync_copy(sc_v, sc.at[wid * B + b])

        for ci in range(ND):
            c = wid * ND + ci
            pltpu.sync_copy(fm.at[b, c], fm_v.at[pl.ds(0, HW)])

            def gather_body(i, _):
                base = i * (4 * L)
                for u in range(4):
                    off = base + u * L
                    iv = idx_v[pl.ds(off, L)]
                    vals = plsc.load_gather(fm_v, [iv])
                    out_v[pl.ds(off, L)] = vals * scalev
                return 0

            lax.fori_loop(0, HW // (4 * L), gather_body, 0)
            pltpu.sync_copy(out_v, sf.at[b, c])


def kernel(feature_map, dist, sin_angles, cos_angles, nd_sampling):
    assert feature_map.shape == (B, C, H, W)
    fm = feature_map.reshape(B, C, HW)
    dist2 = dist.reshape(B, HW)
    nd_scale = jnp.asarray(nd_sampling, jnp.float32) / jnp.float32(ND)
    consts = jnp.stack([
        jnp.broadcast_to(sin_angles.reshape(K, 1), (K, L)),
        jnp.broadcast_to(cos_angles.reshape(K, 1), (K, L)),
        jnp.broadcast_to(nd_scale.reshape(1, 1), (K, L)),
    ], axis=1).astype(jnp.float32).reshape(K * 3 * L)

    mesh = plsc.VectorSubcoreMesh(core_axis_name="c", subcore_axis_name="s")
    run = functools.partial(
        pl.kernel,
        mesh=mesh,
        compiler_params=pltpu.CompilerParams(needs_layout_passes=False),
        out_type=[
            jax.ShapeDtypeStruct((B, C, HW), jnp.float32),
            jax.ShapeDtypeStruct((K * B, 2 * HW), jnp.float32),
        ],
        scratch_types=[
            pltpu.VMEM((HW + 128,), jnp.float32),
            pltpu.VMEM((HW,), jnp.float32),
            pltpu.VMEM((HW,), jnp.int32),
            pltpu.VMEM((2 * HW,), jnp.float32),
            pltpu.VMEM((HW,), jnp.float32),
            pltpu.VMEM((K * 3 * L,), jnp.float32),
        ],
    )(_sc_body)
    sf_flat, sc_flat = run(fm, dist2, consts)
    sf = sf_flat.reshape(B, C, H, W)
    sc = sc_flat.reshape(K * B, H, W, 2)
    return (sf, sc)


# speedup vs baseline: 2.5433x; 1.1344x over previous
import functools

import jax
import jax.numpy as jnp
from jax import lax
from jax.experimental import pallas as pl
from jax.experimental.pallas import tpu as pltpu
from jax.experimental.pallas import tpu_sc as plsc

B, C, H, W = 2, 384, 128, 128
K = 32
ND = C // K
HW = H * W
L = 16
MAGIC = float(1.5 * 2.0**23)


def _sc_body(fm, dist, consts, sf, sc,
             fm_a, fm_b, out_a, out_b, idx_v, sc_v, const_v,
             sem_fa, sem_fb, sem_oa, sem_ob, sem_sc):
    wid = lax.axis_index("s") * 2 + lax.axis_index("c")

    pltpu.sync_copy(consts, const_v)
    zero = jnp.zeros((L,), jnp.float32)
    fm_a[pl.ds(HW, L)] = zero
    fm_b[pl.ds(HW, L)] = zero

    cbase = wid * (3 * L)
    sinv = const_v[pl.ds(cbase, L)]
    cosv = const_v[pl.ds(cbase + L, L)]
    scalev = const_v[pl.ds(cbase + 2 * L, L)]
    lanes = lax.iota(jnp.int32, L)
    lanes_f = lanes.astype(jnp.float32)

    fm_bufs = (fm_a, fm_b)
    fm_sems = (sem_fa, sem_fb)
    out_bufs = (out_a, out_b)
    out_sems = (sem_oa, sem_ob)
    out_pending = [False, False]

    for b in range(B):
        c0 = wid * ND
        if out_pending[0]:
            pltpu.make_async_copy(out_a, sf.at[0, c0], sem_oa).wait()
            out_pending[0] = False
        pltpu.sync_copy(dist.at[b], out_a)
        pltpu.async_copy(fm.at[b, c0], fm_a.at[pl.ds(0, HW)], sem_fa)
        pltpu.async_copy(fm.at[b, c0 + 1], fm_b.at[pl.ds(0, HW)], sem_fb)
        if b > 0:
            pltpu.make_async_copy(sc_v, sc.at[wid * B], sem_sc).wait()

        def idx_body(y, _):
            yf = y.astype(jnp.float32)
            for j in range(W // L):
                off = y * W + j * L
                d = out_a[pl.ds(off, L)]
                xf = lanes_f + jnp.float32(j * L)
                fx = xf + cosv * d
                fy = yf + sinv * d
                scx = (fx / jnp.float32(W - 1)) * 2.0 - 1.0
                scy = (fy / jnp.float32(H - 1)) * 2.0 - 1.0
                ixf = (scx + 1.0) * jnp.float32(W) * 0.5 - 0.5
                iyf = (scy + 1.0) * jnp.float32(H) * 0.5 - 0.5
                rx = (ixf + MAGIC) - MAGIC
                ry = (iyf + MAGIC) - MAGIC
                valid = ((rx >= 0.0) & (rx <= jnp.float32(W - 1))
                         & (ry >= 0.0) & (ry <= jnp.float32(H - 1)))
                cx = jnp.minimum(jnp.maximum(rx, 0.0), jnp.float32(W - 1))
                cy = jnp.minimum(jnp.maximum(ry, 0.0), jnp.float32(H - 1))
                idxf = cy * jnp.float32(W) + cx
                idxf = jnp.where(valid, idxf, jnp.float32(HW))
                idx_v[pl.ds(off, L)] = idxf.astype(jnp.int32)
                p = (off + lanes) * 2
                plsc.store_scatter(sc_v, [p], scx)
                plsc.store_scatter(sc_v, [p + 1], scy)
            return 0

        lax.fori_loop(0, H, idx_body, 0)
        pltpu.async_copy(sc_v, sc.at[wid * B + b], sem_sc)

        for ci in range(ND):
            par = ci % 2
            cur = fm_bufs[par]
            pltpu.make_async_copy(fm.at[b, c0 + ci], cur.at[pl.ds(0, HW)],
                                  fm_sems[par]).wait()
            out = out_bufs[par]
            if out_pending[par]:
                pltpu.make_async_copy(out, sf.at[b, c0], out_sems[par]).wait()
                out_pending[par] = False

            def gather_body(i, _):
                base = i * (4 * L)
                for u in range(4):
                    o = base + u * L
                    iv = idx_v[pl.ds(o, L)]
                    vals = plsc.load_gather(cur, [iv])
                    out[pl.ds(o, L)] = vals * scalev
                return 0

            lax.fori_loop(0, HW // (4 * L), gather_body, 0)
            pltpu.async_copy(out, sf.at[b, c0 + ci], out_sems[par])
            out_pending[par] = True
            if ci + 2 < ND:
                pltpu.async_copy(fm.at[b, c0 + ci + 2],
                                 cur.at[pl.ds(0, HW)], fm_sems[par])

    if out_pending[0]:
        pltpu.make_async_copy(out_a, sf.at[0, 0], sem_oa).wait()
    if out_pending[1]:
        pltpu.make_async_copy(out_b, sf.at[0, 0], sem_ob).wait()
    pltpu.make_async_copy(sc_v, sc.at[0], sem_sc).wait()


def kernel(feature_map, dist, sin_angles, cos_angles, nd_sampling):
    assert feature_map.shape == (B, C, H, W)
    fm = feature_map.reshape(B, C, HW)
    dist2 = dist.reshape(B, HW)
    nd_scale = jnp.asarray(nd_sampling, jnp.float32) / jnp.float32(ND)
    consts = jnp.stack([
        jnp.broadcast_to(sin_angles.reshape(K, 1), (K, L)),
        jnp.broadcast_to(cos_angles.reshape(K, 1), (K, L)),
        jnp.broadcast_to(nd_scale.reshape(1, 1), (K, L)),
    ], axis=1).astype(jnp.float32).reshape(K * 3 * L)

    mesh = plsc.VectorSubcoreMesh(core_axis_name="c", subcore_axis_name="s")
    run = functools.partial(
        pl.kernel,
        mesh=mesh,
        compiler_params=pltpu.CompilerParams(needs_layout_passes=False),
        out_type=[
            jax.ShapeDtypeStruct((B, C, HW), jnp.float32),
            jax.ShapeDtypeStruct((K * B, 2 * HW), jnp.float32),
        ],
        scratch_types=[
            pltpu.VMEM((HW + 128,), jnp.float32),
            pltpu.VMEM((HW + 128,), jnp.float32),
            pltpu.VMEM((HW,), jnp.float32),
            pltpu.VMEM((HW,), jnp.float32),
            pltpu.VMEM((HW,), jnp.int32),
            pltpu.VMEM((2 * HW,), jnp.float32),
            pltpu.VMEM((K * 3 * L,), jnp.float32),
            pltpu.SemaphoreType.DMA,
            pltpu.SemaphoreType.DMA,
            pltpu.SemaphoreType.DMA,
            pltpu.SemaphoreType.DMA,
            pltpu.SemaphoreType.DMA,
        ],
    )(_sc_body)
    sf_flat, sc_flat = run(fm, dist2, consts)
    sf = sf_flat.reshape(B, C, H, W)
    sc = sc_flat.reshape(K * B, H, W, 2)
    return (sf, sc)


# speedup vs baseline: 4.2093x; 1.6551x over previous
import functools

import jax
import jax.numpy as jnp
from jax import lax
from jax.experimental import pallas as pl
from jax.experimental.pallas import tpu as pltpu
from jax.experimental.pallas import tpu_sc as plsc

B, C, H, W = 2, 384, 128, 128
K = 32
ND = C // K
HW = H * W
L = 16
MAGIC = float(1.5 * 2.0**23)


def _sc_body(fm, dist, consts, sf, sc,
             fm_a, fm_b, out_a, out_b, idx_v, sc_v, const_v,
             sem_fa, sem_fb, sem_oa, sem_ob, sem_sc):
    wid = lax.axis_index("s") * 2 + lax.axis_index("c")

    pltpu.sync_copy(consts, const_v)
    zero = jnp.zeros((L,), jnp.float32)
    fm_a[pl.ds(HW, L)] = zero
    fm_b[pl.ds(HW, L)] = zero

    cbase = wid * (3 * L)
    sinv = const_v[pl.ds(cbase, L)]
    cosv = const_v[pl.ds(cbase + L, L)]
    scalev = const_v[pl.ds(cbase + 2 * L, L)]
    lanes = lax.iota(jnp.int32, L)
    lanes_f = lanes.astype(jnp.float32)

    fm_bufs = (fm_a, fm_b)
    fm_sems = (sem_fa, sem_fb)
    out_bufs = (out_a, out_b)
    out_sems = (sem_oa, sem_ob)
    out_pending = [False, False]

    for b in range(B):
        c0 = wid * ND
        if out_pending[0]:
            pltpu.make_async_copy(out_a, sf.at[0, c0], sem_oa).wait()
            out_pending[0] = False
        pltpu.sync_copy(dist.at[b], out_a)
        pltpu.async_copy(fm.at[b, c0], fm_a.at[pl.ds(0, HW)], sem_fa)
        pltpu.async_copy(fm.at[b, c0 + 1], fm_b.at[pl.ds(0, HW)], sem_fb)
        if b > 0:
            pltpu.make_async_copy(sc_v, sc.at[wid * B], sem_sc).wait()

        @plsc.parallel_loop(0, H, 1, unroll=2)
        def idx_body(y):
            yf = y.astype(jnp.float32)
            for j in range(W // L):
                off = y * W + j * L
                d = out_a[pl.ds(off, L)]
                xf = lanes_f + jnp.float32(j * L)
                fx = xf + cosv * d
                fy = yf + sinv * d
                scx = (fx / jnp.float32(W - 1)) * 2.0 - 1.0
                scy = (fy / jnp.float32(H - 1)) * 2.0 - 1.0
                ixf = (scx + 1.0) * jnp.float32(W) * 0.5 - 0.5
                iyf = (scy + 1.0) * jnp.float32(H) * 0.5 - 0.5
                rx = (ixf + MAGIC) - MAGIC
                ry = (iyf + MAGIC) - MAGIC
                valid = ((rx >= 0.0) & (rx <= jnp.float32(W - 1))
                         & (ry >= 0.0) & (ry <= jnp.float32(H - 1)))
                cx = jnp.minimum(jnp.maximum(rx, 0.0), jnp.float32(W - 1))
                cy = jnp.minimum(jnp.maximum(ry, 0.0), jnp.float32(H - 1))
                idxf = cy * jnp.float32(W) + cx
                idxf = jnp.where(valid, idxf, jnp.float32(HW))
                idx_v[pl.ds(off, L)] = idxf.astype(jnp.int32)
                p = (off + lanes) * 2
                plsc.store_scatter(sc_v, [p], scx)
                plsc.store_scatter(sc_v, [p + 1], scy)

        pltpu.async_copy(sc_v, sc.at[wid * B + b], sem_sc)

        for ci in range(ND):
            par = ci % 2
            cur = fm_bufs[par]
            pltpu.make_async_copy(fm.at[b, c0 + ci], cur.at[pl.ds(0, HW)],
                                  fm_sems[par]).wait()
            out = out_bufs[par]
            if out_pending[par]:
                pltpu.make_async_copy(out, sf.at[b, c0], out_sems[par]).wait()
                out_pending[par] = False

            @plsc.parallel_loop(0, HW // L, 1, unroll=8)
            def gather_body(i):
                o = i * L
                iv = idx_v[pl.ds(o, L)]
                vals = plsc.load_gather(cur, [iv])
                out[pl.ds(o, L)] = vals * scalev

            pltpu.async_copy(out, sf.at[b, c0 + ci], out_sems[par])
            out_pending[par] = True
            if ci + 2 < ND:
                pltpu.async_copy(fm.at[b, c0 + ci + 2],
                                 cur.at[pl.ds(0, HW)], fm_sems[par])

    if out_pending[0]:
        pltpu.make_async_copy(out_a, sf.at[0, 0], sem_oa).wait()
    if out_pending[1]:
        pltpu.make_async_copy(out_b, sf.at[0, 0], sem_ob).wait()
    pltpu.make_async_copy(sc_v, sc.at[0], sem_sc).wait()


def kernel(feature_map, dist, sin_angles, cos_angles, nd_sampling):
    assert feature_map.shape == (B, C, H, W)
    fm = feature_map.reshape(B, C, HW)
    dist2 = dist.reshape(B, HW)
    nd_scale = jnp.asarray(nd_sampling, jnp.float32) / jnp.float32(ND)
    consts = jnp.stack([
        jnp.broadcast_to(sin_angles.reshape(K, 1), (K, L)),
        jnp.broadcast_to(cos_angles.reshape(K, 1), (K, L)),
        jnp.broadcast_to(nd_scale.reshape(1, 1), (K, L)),
    ], axis=1).astype(jnp.float32).reshape(K * 3 * L)

    mesh = plsc.VectorSubcoreMesh(core_axis_name="c", subcore_axis_name="s")
    run = functools.partial(
        pl.kernel,
        mesh=mesh,
        compiler_params=pltpu.CompilerParams(needs_layout_passes=False),
        out_type=[
            jax.ShapeDtypeStruct((B, C, HW), jnp.float32),
            jax.ShapeDtypeStruct((K * B, 2 * HW), jnp.float32),
        ],
        scratch_types=[
            pltpu.VMEM((HW + 128,), jnp.float32),
            pltpu.VMEM((HW + 128,), jnp.float32),
            pltpu.VMEM((HW,), jnp.float32),
            pltpu.VMEM((HW,), jnp.float32),
            pltpu.VMEM((HW,), jnp.int32),
            pltpu.VMEM((2 * HW,), jnp.float32),
            pltpu.VMEM((K * 3 * L,), jnp.float32),
            pltpu.SemaphoreType.DMA,
            pltpu.SemaphoreType.DMA,
            pltpu.SemaphoreType.DMA,
            pltpu.SemaphoreType.DMA,
            pltpu.SemaphoreType.DMA,
        ],
    )(_sc_body)
    sf_flat, sc_flat = run(fm, dist2, consts)
    sf = sf_flat.reshape(B, C, H, W)
    sc = sc_flat.reshape(K * B, H, W, 2)
    return (sf, sc)


# speedup vs baseline: 7.6793x; 1.8243x over previous
import functools

import jax
import jax.numpy as jnp
from jax import lax
from jax.experimental import pallas as pl
from jax.experimental.pallas import tpu as pltpu
from jax.experimental.pallas import tpu_sc as plsc

B, C, H, W = 2, 384, 128, 128
K = 32
ND = C // K
HW = H * W
L = 16
MAGIC = float(1.5 * 2.0**23)


def _sc_body(fm, dist, consts, sf, sc,
             fm_a, fm_b, out_a, out_b, idx_v, sc_v, const_v,
             sem_fa, sem_fb, sem_oa, sem_ob, sem_sc):
    wid = lax.axis_index("s") * 2 + lax.axis_index("c")

    pltpu.sync_copy(consts, const_v)
    zero = jnp.zeros((L,), jnp.float32)
    for t in range(W // L):
        fm_a[H, pl.ds(t * L, L)] = zero
        fm_b[H, pl.ds(t * L, L)] = zero

    cbase = wid * (3 * L)
    sinv = const_v[pl.ds(cbase, L)]
    cosv = const_v[pl.ds(cbase + L, L)]
    scalev = const_v[pl.ds(cbase + 2 * L, L)]
    lanes = lax.iota(jnp.int32, L)
    lanes_f = lanes.astype(jnp.float32)

    fm_bufs = (fm_a, fm_b)
    fm_sems = (sem_fa, sem_fb)
    out_bufs = (out_a, out_b)
    out_sems = (sem_oa, sem_ob)
    out_pending = [False, False]

    for b in range(B):
        c0 = wid * ND
        if out_pending[0]:
            pltpu.make_async_copy(out_a, sf.at[0, c0], sem_oa).wait()
            out_pending[0] = False
        pltpu.sync_copy(dist.at[b, 0], out_a)
        pltpu.async_copy(fm.at[b, c0], fm_a.at[pl.ds(0, H)], sem_fa)
        pltpu.async_copy(fm.at[b, c0 + 1], fm_b.at[pl.ds(0, H)], sem_fb)
        if b > 0:
            pltpu.make_async_copy(sc_v, sc.at[wid * B], sem_sc).wait()

        @plsc.parallel_loop(0, H, 1, unroll=2)
        def idx_body(y):
            yf = y.astype(jnp.float32)
            for j2 in range(W // (2 * L)):
                idx_pair = []
                for j in (2 * j2, 2 * j2 + 1):
                    d = out_a[y, pl.ds(j * L, L)]
                    xf = lanes_f + jnp.float32(j * L)
                    fx = xf + cosv * d
                    fy = yf + sinv * d
                    scx = (fx / jnp.float32(W - 1)) * 2.0 - 1.0
                    scy = (fy / jnp.float32(H - 1)) * 2.0 - 1.0
                    ixf = (scx + 1.0) * jnp.float32(W) * 0.5 - 0.5
                    iyf = (scy + 1.0) * jnp.float32(H) * 0.5 - 0.5
                    rx = (ixf + MAGIC) - MAGIC
                    ry = (iyf + MAGIC) - MAGIC
                    valid = ((rx >= 0.0) & (rx <= jnp.float32(W - 1))
                             & (ry >= 0.0) & (ry <= jnp.float32(H - 1)))
                    cx = jnp.minimum(jnp.maximum(rx, 0.0), jnp.float32(W - 1))
                    cy = jnp.minimum(jnp.maximum(ry, 0.0), jnp.float32(H - 1))
                    idxf = cy * jnp.float32(W) + cx
                    idxf = jnp.where(valid, idxf, jnp.float32(HW))
                    idx_pair.append(idxf.astype(jnp.int32))
                    rvec = jnp.full((L,), 2 * y + (j // 4), jnp.int32)
                    cvec = (j * 2 * L) % W + 2 * lanes
                    plsc.store_scatter(sc_v, [rvec, cvec], scx)
                    plsc.store_scatter(sc_v, [rvec, cvec + 1], scy)
                packed = plsc.pack(idx_pair[0], idx_pair[1],
                                   format=plsc.PackFormat.INTERLEAVED,
                                   preferred_element_type=jnp.int16)
                idx_v[pl.ds((y * W) // 2 + j2 * L, L)] = plsc.bitcast(
                    packed, jnp.int32)

        pltpu.async_copy(sc_v, sc.at[wid * B + b], sem_sc)

        for ci in range(ND):
            par = ci % 2
            cur = fm_bufs[par]
            pltpu.make_async_copy(fm.at[b, c0 + ci], cur.at[pl.ds(0, H)],
                                  fm_sems[par]).wait()
            out = out_bufs[par]
            if out_pending[par]:
                pltpu.make_async_copy(out, sf.at[b, c0], out_sems[par]).wait()
                out_pending[par] = False

            @plsc.parallel_loop(0, HW // (2 * L), 1, unroll=4)
            def gather_body(i):
                r = i >> 2
                cb = (i & 3) * 2 * L
                packed = plsc.bitcast(idx_v[pl.ds(i * L, L)], jnp.int16)
                iva, ivb = plsc.unpack(packed,
                                       format=plsc.PackFormat.INTERLEAVED,
                                       preferred_element_type=jnp.int32)
                va = plsc.load_gather(cur, [iva >> 7, iva & (W - 1)])
                vb = plsc.load_gather(cur, [ivb >> 7, ivb & (W - 1)])
                out[r, pl.ds(cb, L)] = va * scalev
                out[r, pl.ds(cb + L, L)] = vb * scalev

            pltpu.async_copy(out, sf.at[b, c0 + ci], out_sems[par])
            out_pending[par] = True
            if ci + 2 < ND:
                pltpu.async_copy(fm.at[b, c0 + ci + 2],
                                 cur.at[pl.ds(0, H)], fm_sems[par])

    if out_pending[0]:
        pltpu.make_async_copy(out_a, sf.at[0, 0], sem_oa).wait()
    if out_pending[1]:
        pltpu.make_async_copy(out_b, sf.at[0, 0], sem_ob).wait()
    pltpu.make_async_copy(sc_v, sc.at[0], sem_sc).wait()


def kernel(feature_map, dist, sin_angles, cos_angles, nd_sampling):
    assert feature_map.shape == (B, C, H, W)
    nd_scale = jnp.asarray(nd_sampling, jnp.float32) / jnp.float32(ND)
    consts = jnp.stack([
        jnp.broadcast_to(sin_angles.reshape(K, 1), (K, L)),
        jnp.broadcast_to(cos_angles.reshape(K, 1), (K, L)),
        jnp.broadcast_to(nd_scale.reshape(1, 1), (K, L)),
    ], axis=1).astype(jnp.float32).reshape(K * 3 * L)

    mesh = plsc.VectorSubcoreMesh(core_axis_name="c", subcore_axis_name="s")
    run = functools.partial(
        pl.kernel,
        mesh=mesh,
        compiler_params=pltpu.CompilerParams(needs_layout_passes=False),
        out_type=[
            jax.ShapeDtypeStruct((B, C, H, W), jnp.float32),
            jax.ShapeDtypeStruct((K * B, 2 * H, W), jnp.float32),
        ],
        scratch_types=[
            pltpu.VMEM((H + 1, W), jnp.float32),
            pltpu.VMEM((H + 1, W), jnp.float32),
            pltpu.VMEM((H, W), jnp.float32),
            pltpu.VMEM((H, W), jnp.float32),
            pltpu.VMEM((HW // 2,), jnp.int32),
            pltpu.VMEM((2 * H, W), jnp.float32),
            pltpu.VMEM((K * 3 * L,), jnp.float32),
            pltpu.SemaphoreType.DMA,
            pltpu.SemaphoreType.DMA,
            pltpu.SemaphoreType.DMA,
            pltpu.SemaphoreType.DMA,
            pltpu.SemaphoreType.DMA,
        ],
    )(_sc_body)
    sf, sc_flat = run(feature_map, dist, consts)
    sc = sc_flat.reshape(K * B, H, W, 2)
    return (sf, sc)
